# baseline (speedup 1.0000x reference)
import functools

import jax
import jax.numpy as jnp
from jax import lax
from jax.experimental import pallas as pl
from jax.experimental.pallas import tpu as pltpu
from jax.experimental.pallas import tpu_sc as plsc

EMB = 64
BATCH = 16384

_info = plsc.get_sparse_core_info()
_NC, _NS = _info.num_cores, _info.num_subcores
_NW = _NC * _NS
_BPW = BATCH // _NW
_K = 32
_G = _BPW // _K


@functools.partial(
    pl.kernel,
    out_type=jax.ShapeDtypeStruct((BATCH, EMB), jnp.float32),
    mesh=plsc.VectorSubcoreMesh(core_axis_name="c", subcore_axis_name="s"),
    scratch_types=[
        pltpu.VMEM_SHARED((_NS, _BPW), jnp.int32),
        pltpu.SMEM((_BPW,), jnp.int32),
        pltpu.SemaphoreType.DMA,
    ],
    compiler_params=pltpu.CompilerParams(use_tc_tiling_on_sc=True),
)
def _gather(table_hbm, idx_hbm, out_hbm, idx_sh, idx_s, sem):
    wid = lax.axis_index("s") * _NC + lax.axis_index("c")
    sid = lax.axis_index("s")
    base = wid * _BPW
    pltpu.sync_copy(idx_hbm.at[wid], idx_sh.at[sid])
    pltpu.sync_copy(idx_sh.at[sid], idx_s)

    def fire(g):
        for k in range(_K):
            j = g * _K + k
            pltpu.async_copy(table_hbm.at[idx_s[j]], out_hbm.at[base + j], sem)

    def drain_one_group():
        pltpu.make_async_copy(
            table_hbm.at[pl.ds(0, _K)],
            out_hbm.at[pl.ds(base, _K)],
            sem,
        ).wait()

    fire(0)

    def body(g, carry):
        fire(g)
        drain_one_group()
        return carry

    lax.fori_loop(1, _G, body, 0)
    drain_one_group()


def kernel(node_ids, table):
    idx = node_ids.astype(jnp.int32).reshape(_NW, _BPW)
    return _gather(table, idx)


# speedup vs baseline: 1.4905x; 1.4905x over previous
import functools

import jax
import jax.numpy as jnp
from jax import lax
from jax.experimental import pallas as pl
from jax.experimental.pallas import tpu as pltpu
from jax.experimental.pallas import tpu_sc as plsc

EMB = 64
BATCH = 16384
NB = 500001

_info = plsc.get_sparse_core_info()
_NC, _NS = _info.num_cores, _info.num_subcores
_NW = _NC * _NS
_BPW = BATCH // _NW
_CHUNK = 128
_NCHUNK = _BPW // _CHUNK


@functools.partial(
    pl.kernel,
    out_type=jax.ShapeDtypeStruct((BATCH, 128), jnp.float32),
    mesh=plsc.VectorSubcoreMesh(core_axis_name="c", subcore_axis_name="s"),
    scratch_types=[
        pltpu.VMEM((_NCHUNK, _CHUNK), jnp.int32),
        pltpu.VMEM((_NCHUNK, _CHUNK, 128), jnp.float32),
        pltpu.SemaphoreType.DMA,
    ],
    compiler_params=pltpu.CompilerParams(use_tc_tiling_on_sc=True),
)
def _gather(table_hbm, idx_hbm, out_hbm, idx_v, rows_v, sem):
    wid = lax.axis_index("s") * _NC + lax.axis_index("c")
    base = wid * _BPW
    pltpu.sync_copy(idx_hbm.at[wid], idx_v)
    copies = [
        pltpu.async_copy(table_hbm.at[idx_v.at[c]], rows_v.at[c], sem)
        for c in range(_NCHUNK)
    ]
    for cp in copies:
        cp.wait()
    for c in range(_NCHUNK):
        pltpu.sync_copy(rows_v.at[c], out_hbm.at[pl.ds(base + c * _CHUNK, _CHUNK)])


def kernel(node_ids, table):
    idx = node_ids.astype(jnp.int32).reshape(_NW, _NCHUNK, _CHUNK)
    tbl = jnp.pad(table, ((0, 7), (0, 64)))
    out = _gather(tbl, idx)
    return out[:, :EMB]
